# baseline (speedup 1.0000x reference)
import functools

import jax
import jax.numpy as jnp
from jax import lax
from jax.experimental import pallas as pl
from jax.experimental.pallas import tpu as pltpu
from jax.experimental.pallas import tpu_sc as plsc

_VOCAB = 1000000
_D = 64
_BATCH = 4096
_HIST = 200

_NC, _NS = 2, 16
_NW = _NC * _NS
_RPW = _BATCH // _NW
_TPW = _RPW * _HIST
_K = _RPW

_mesh = plsc.VectorSubcoreMesh(
    core_axis_name="c", subcore_axis_name="s",
    num_cores=_NC, num_subcores=_NS)


@functools.partial(
    pl.kernel,
    out_type=jax.ShapeDtypeStruct((_HIST, _D, _BATCH), jnp.float32),
    mesh=_mesh,
    scratch_types=[
        pltpu.VMEM((_TPW,), jnp.int32),
        pltpu.VMEM((2, _K, _D), jnp.float32),
        pltpu.VMEM((2, _D, _K), jnp.float32),
        pltpu.SemaphoreType.DMA,
        pltpu.SemaphoreType.DMA,
        pltpu.SemaphoreType.DMA,
        pltpu.SemaphoreType.DMA,
    ],
    compiler_params=pltpu.CompilerParams(
        use_tc_tiling_on_sc=True, needs_layout_passes=False),
)
def _emb_gather(xf_hbm, tbl_hbm, out_hbm, idx_v, buf, tbuf, g0, g1, o0, o1):
    gsem = (g0, g1)
    osem = (o0, o1)
    wid = lax.axis_index("s") * _NC + lax.axis_index("c")
    b0 = wid * _RPW

    pltpu.sync_copy(xf_hbm.at[pl.ds(wid * _TPW, _TPW)], idx_v)

    lane = lax.iota(jnp.int32, 16)

    def issue(t, p):
        for q in range(8):
            pos = (q * 16 + lane) * _HIST + t
            vs = plsc.load_gather(idx_v, [pos])
            for j in range(16):
                pltpu.make_async_copy(
                    tbl_hbm.at[pl.ds(vs[j], 1)],
                    buf.at[p].at[pl.ds(q * 16 + j, 1)], gsem[p]).start()

    def drain_desc(p):
        return pltpu.make_async_copy(
            tbl_hbm.at[pl.ds(0, _K)], buf.at[p], gsem[p])

    def out_desc(t, p):
        return pltpu.make_async_copy(
            tbuf.at[p], out_hbm.at[t, :, pl.ds(b0, _RPW)], osem[p])

    def transpose(p):
        rows_qs = [q * 16 + lane for q in range(8)]

        def dloop(d, c):
            dv = jnp.full((16,), d, jnp.int32)
            for q in range(8):
                vals = plsc.load_gather(buf.at[p], [rows_qs[q], dv])
                plsc.store_scatter(tbuf.at[p], [dv, rows_qs[q]], vals)
            return c

        lax.fori_loop(0, _D, dloop, 0)

    for p in range(2):
        issue(p, p)

    def group(g, carry):
        for p in range(2):
            t = g * 2 + p
            drain_desc(p).wait()

            @pl.when(t >= 2)
            def _():
                out_desc(t - 2, p).wait()

            transpose(p)
            out_desc(t, p).start()
            issue(t + 2, p)
        return carry

    lax.fori_loop(0, _HIST // 2 - 1, group, 0)

    for p in range(2):
        t = _HIST - 2 + p
        drain_desc(p).wait()
        out_desc(t - 2, p).wait()
        transpose(p)
        out_desc(t, p).start()
    for p in range(2):
        out_desc(_HIST - 2 + p, p).wait()


def kernel(x, table):
    xf = x.astype(jnp.int32).reshape(_BATCH * _HIST)
    out_p = _emb_gather(xf, table)
    return jnp.transpose(out_p, (2, 0, 1))
